# Initial kernel scaffold; baseline (speedup 1.0000x reference)
import jax
import jax.numpy as jnp
from jax.experimental import pallas as pl


def kernel(x, weight):
    raise NotImplementedError("write your pallas kernel here")


# speedup vs baseline: 1.2753x; 1.2753x over previous
import functools

import jax
import jax.numpy as jnp
from jax import lax
from jax.experimental import pallas as pl
from jax.experimental.pallas import tpu as pltpu
from jax.experimental.pallas import tpu_sc as plsc


def _tc_argmin_body(x_ref, w_ref, o_ref):
    xb = x_ref[0]
    w = w_ref[...]
    xw = lax.dot_general(xb, w, (((0,), (0,)), ((), ())),
                         preferred_element_type=jnp.float32)
    e2 = jnp.sum(w * w, axis=0)[None, :]
    m = e2 - 2.0 * xw
    rowmin = jnp.min(m, axis=1, keepdims=True)
    K = m.shape[1]
    ks = lax.broadcasted_iota(jnp.int32, m.shape, 1)
    idx = jnp.min(jnp.where(m == rowmin, ks, K), axis=1)
    o_ref[0, 0] = idx.astype(jnp.int32)


def _sc_gather_body(w_hbm, idx_hbm, out_hbm, w_v, idx_v, out_v):
    info = plsc.get_sparse_core_info()
    nc = info.num_cores
    wid = lax.axis_index("s") * nc + lax.axis_index("c")
    b = wid // 2
    d0 = (wid % 2) * 32
    pltpu.sync_copy(idx_hbm.at[b], idx_v)
    pltpu.sync_copy(w_hbm.at[pl.ds(d0, 32)], w_v)

    def chunk(c, carry):
        iv = idx_v[pl.ds(c * 16, 16)]
        for dl in range(32):
            dv = jnp.full((16,), dl, jnp.int32)
            out_v[dl, pl.ds(c * 16, 16)] = plsc.load_gather(w_v, [dv, iv])
        return carry

    lax.fori_loop(0, idx_v.shape[0] // 16, chunk, 0)
    pltpu.sync_copy(out_v, out_hbm.at[b, pl.ds(d0, 32)])


def kernel(x, weight):
    B, d, H, W = x.shape
    K = weight.shape[1]
    HW = H * W
    xr = x.reshape(B, d, HW)

    idx3 = pl.pallas_call(
        _tc_argmin_body,
        grid=(B,),
        in_specs=[
            pl.BlockSpec((1, d, HW), lambda i: (i, 0, 0)),
            pl.BlockSpec((d, K), lambda i: (0, 0)),
        ],
        out_specs=pl.BlockSpec((1, 1, HW), lambda i: (i, 0, 0)),
        out_shape=jax.ShapeDtypeStruct((B, 1, HW), jnp.int32),
    )(xr, weight)
    idx = idx3.reshape(B, HW)

    sc_gather = pl.kernel(
        _sc_gather_body,
        out_type=jax.ShapeDtypeStruct((B, d, HW), jnp.float32),
        mesh=plsc.VectorSubcoreMesh(core_axis_name="c", subcore_axis_name="s"),
        scratch_types=[
            pltpu.VMEM((32, K), jnp.float32),
            pltpu.VMEM((HW,), jnp.int32),
            pltpu.VMEM((32, HW), jnp.float32),
        ],
        compiler_params=pltpu.CompilerParams(
            use_tc_tiling_on_sc=False, needs_layout_passes=False),
    )
    quant = sc_gather(weight, idx)

    return quant.reshape(B, d, H, W), idx.reshape(B, H, W)


# speedup vs baseline: 1.6334x; 1.2807x over previous
import functools

import jax
import jax.numpy as jnp
from jax import lax
from jax.experimental import pallas as pl
from jax.experimental.pallas import tpu as pltpu
from jax.experimental.pallas import tpu_sc as plsc


def _tc_argmin_body(x_ref, w_ref, o_ref):
    xb = x_ref[0]
    w = w_ref[...]
    HW = xb.shape[1]
    del HW
    e2 = jnp.sum(w * w, axis=0)[:, None]
    mm = lax.dot_general(w * -2.0, xb, (((0,), (0,)), ((), ())),
                         preferred_element_type=jnp.float32)
    m = e2 + mm
    colmin = jnp.min(m, axis=0, keepdims=True)
    K = m.shape[0]
    ks = lax.broadcasted_iota(jnp.int32, m.shape, 0)
    idx = jnp.min(jnp.where(m == colmin, ks, K), axis=0)
    o_ref[0, 0] = idx.astype(jnp.int32)


def _sc_gather_body(w_hbm, idx_hbm, out_hbm, w_v, idx_v, out_v):
    info = plsc.get_sparse_core_info()
    nc = info.num_cores
    wid = lax.axis_index("s") * nc + lax.axis_index("c")
    b = wid // 2
    d0 = (wid % 2) * 32
    pltpu.sync_copy(idx_hbm.at[b], idx_v)
    pltpu.sync_copy(w_hbm.at[pl.ds(d0, 32)], w_v)

    def chunk(c, carry):
        iv = idx_v[pl.ds(c * 16, 16)]
        for dl in range(32):
            dv = jnp.full((16,), dl, jnp.int32)
            out_v[dl, pl.ds(c * 16, 16)] = plsc.load_gather(w_v, [dv, iv])
        return carry

    lax.fori_loop(0, idx_v.shape[0] // 16, chunk, 0)
    pltpu.sync_copy(out_v, out_hbm.at[b, pl.ds(d0, 32)])


def kernel(x, weight):
    B, d, H, W = x.shape
    K = weight.shape[1]
    HW = H * W
    xr = x.reshape(B, d, HW)

    idx3 = pl.pallas_call(
        _tc_argmin_body,
        grid=(B,),
        in_specs=[
            pl.BlockSpec((1, d, HW), lambda i: (i, 0, 0)),
            pl.BlockSpec((d, K), lambda i: (0, 0)),
        ],
        out_specs=pl.BlockSpec((1, 1, HW), lambda i: (i, 0, 0)),
        out_shape=jax.ShapeDtypeStruct((B, 1, HW), jnp.int32),
    )(xr, weight)
    idx = idx3.reshape(B, HW)

    sc_gather = pl.kernel(
        _sc_gather_body,
        out_type=jax.ShapeDtypeStruct((B, d, HW), jnp.float32),
        mesh=plsc.VectorSubcoreMesh(core_axis_name="c", subcore_axis_name="s"),
        scratch_types=[
            pltpu.VMEM((32, K), jnp.float32),
            pltpu.VMEM((HW,), jnp.int32),
            pltpu.VMEM((32, HW), jnp.float32),
        ],
        compiler_params=pltpu.CompilerParams(
            use_tc_tiling_on_sc=False, needs_layout_passes=False),
    )
    quant = sc_gather(weight, idx)

    return quant.reshape(B, d, H, W), idx.reshape(B, H, W)
